# baseline (speedup 1.0000x reference)
import functools

import jax
import jax.numpy as jnp
from jax import lax
from jax.experimental import pallas as pl
from jax.experimental.pallas import tpu as pltpu
from jax.experimental.pallas import tpu_sc as plsc

NC = 2
NS = 16
NW = NC * NS
C = 128


def _sc_mesh():
  return plsc.VectorSubcoreMesh(
      core_axis_name="c", subcore_axis_name="s", num_cores=NC, num_subcores=NS
  )


def _make_gather(epad, k, d, dtype=jnp.float32):
  @functools.partial(
      pl.kernel,
      out_type=jax.ShapeDtypeStruct((epad, d), dtype),
      mesh=_sc_mesh(),
      scratch_types=[
          pltpu.VMEM((C,), jnp.int32),
          pltpu.VMEM((C, d), dtype),
          pltpu.SemaphoreType.DMA,
      ],
      compiler_params=pltpu.CompilerParams(use_tc_tiling_on_sc=False),
  )
  def gather(table_hbm, idx_hbm, out_hbm, idx_v, rows_v, sem):
    cid = lax.axis_index("c")
    sid = lax.axis_index("s")
    w = cid * NS + sid

    def body(j, _):
      pltpu.sync_copy(idx_hbm.at[w, j], idx_v)
      pltpu.async_copy(table_hbm.at[idx_v], rows_v, sem).wait()
      base = (w * k + j) * C
      pltpu.sync_copy(rows_v, out_hbm.at[pl.ds(base, C), :])
      return 0

    lax.fori_loop(0, k, body, 0)

  return gather


def _make_scatter_add(epad, k, npad):
  rpt = npad // NS

  @functools.partial(
      pl.kernel,
      out_type=jax.ShapeDtypeStruct((NC, npad, 32), jnp.float32),
      mesh=_sc_mesh(),
      scratch_types=[
          pltpu.VMEM((C,), jnp.int32),
          pltpu.VMEM((C, 32), jnp.float32),
          pltpu.VMEM_SHARED((npad, 32), jnp.float32),
      ],
      compiler_params=pltpu.CompilerParams(use_tc_tiling_on_sc=False),
  )
  def scatter(msg_hbm, dst_hbm, zeros_hbm, out_hbm, idx_v, rows_v, acc):
    cid = lax.axis_index("c")
    sid = lax.axis_index("s")
    w = cid * NS + sid

    pltpu.sync_copy(zeros_hbm.at[pl.ds(sid * rpt, rpt), :],
                    acc.at[pl.ds(sid * rpt, rpt), :])
    plsc.subcore_barrier()

    def body(j, _):
      pltpu.sync_copy(dst_hbm.at[w, j], idx_v)
      base = (w * k + j) * C
      pltpu.sync_copy(msg_hbm.at[pl.ds(base, C), :], rows_v)
      pltpu.sync_copy(rows_v, acc.at[idx_v], add=True)
      return 0

    lax.fori_loop(0, k, body, 0)
    plsc.subcore_barrier()
    pltpu.sync_copy(acc.at[pl.ds(sid * rpt, rpt), :],
                    out_hbm.at[cid, pl.ds(sid * rpt, rpt), :])

  return scatter


def _msg_body(nh, ea_ref, hg_ref, w1_ref, b1_ref, w2m_ref, b2m_ref, out_ref):
  ea = ea_ref[...]
  t = jnp.maximum(
      jnp.dot(ea, w1_ref[...], preferred_element_type=jnp.float32)
      + b1_ref[...], 0.0)
  hg = hg_ref[...][:, :nh]
  u = jnp.concatenate([t[:, kk : kk + 1] * hg for kk in range(16)], axis=1)
  out_ref[...] = (
      jnp.dot(u, w2m_ref[...], preferred_element_type=jnp.float32)
      + jnp.dot(hg, b2m_ref[...], preferred_element_type=jnp.float32))


def _tc_msg(ea, hg, w1, b1, w2, b2, nh, bt):
  epad = ea.shape[0]
  grid = epad // bt
  w2m = w2.reshape(16, nh, 32).reshape(16 * nh, 32)
  b2m = b2.reshape(nh, 32)
  return pl.pallas_call(
      functools.partial(_msg_body, nh),
      grid=(grid,),
      in_specs=[
          pl.BlockSpec((bt, 2), lambda i: (i, 0)),
          pl.BlockSpec((bt, hg.shape[1]), lambda i: (i, 0)),
          pl.BlockSpec((2, 16), lambda i: (0, 0)),
          pl.BlockSpec((1, 16), lambda i: (0, 0)),
          pl.BlockSpec((16 * nh, 32), lambda i: (0, 0)),
          pl.BlockSpec((nh, 32), lambda i: (0, 0)),
      ],
      out_specs=pl.BlockSpec((bt, 32), lambda i: (i, 0)),
      out_shape=jax.ShapeDtypeStruct((epad, 32), jnp.float32),
  )(ea, hg, w1, b1.reshape(1, -1), w2m, b2m)


def _node_body(relu_out, p_ref, h_ref, r_ref, b_ref, wo_ref, bo_ref, out_ref):
  agg = p_ref[0] + p_ref[1]
  h = jnp.maximum(
      agg + jnp.dot(h_ref[...], r_ref[...], preferred_element_type=jnp.float32)
      + b_ref[...], 0.0)
  if relu_out:
    out_ref[...] = h
  else:
    out_ref[...] = (
        jnp.dot(h, wo_ref[...], preferred_element_type=jnp.float32)
        + bo_ref[...])


def _tc_node(partials, h, root, bias, w_out, b_out, relu_out, bn):
  npad = h.shape[0]
  dh = h.shape[1]
  dout = 32 if relu_out else w_out.shape[1]
  return pl.pallas_call(
      functools.partial(_node_body, relu_out),
      grid=(npad // bn,),
      in_specs=[
          pl.BlockSpec((2, bn, 32), lambda i: (0, i, 0)),
          pl.BlockSpec((bn, dh), lambda i: (i, 0)),
          pl.BlockSpec((dh, 32), lambda i: (0, 0)),
          pl.BlockSpec((1, 32), lambda i: (0, 0)),
          pl.BlockSpec(w_out.shape, lambda i: (0, 0)),
          pl.BlockSpec((1, w_out.shape[1]), lambda i: (0, 0)),
      ],
      out_specs=pl.BlockSpec((bn, dout), lambda i: (i, 0)),
      out_shape=jax.ShapeDtypeStruct((npad, dout), jnp.float32),
  )(partials, h, root, bias.reshape(1, -1), w_out, b_out.reshape(1, -1))


def kernel(x, edge_index, edge_attr,
           nn1_w1, nn1_b1, nn1_w2, nn1_b2, root1, bias1,
           nn2_w1, nn2_b1, nn2_w2, nn2_b2, root2, bias2,
           fc2_w, fc2_b):
  n = x.shape[0]
  e = edge_attr.shape[0]
  k = -(-e // (NW * C))
  epad = NW * k * C
  npad = -(-(n + 1) // 1024) * 1024

  src = edge_index[0]
  dst = edge_index[1]
  src3 = jnp.zeros((epad,), jnp.int32).at[:e].set(src).reshape(NW, k, C)
  dst3 = jnp.full((epad,), n, jnp.int32).at[:e].set(dst).reshape(NW, k, C)
  ea_pad = jnp.zeros((epad, 2), jnp.float32).at[:e].set(edge_attr)
  x16 = jnp.zeros((n, 16), jnp.float32).at[:, :2].set(x)
  x_pad = jnp.zeros((npad, 2), jnp.float32).at[:n].set(x)
  zacc = jnp.zeros((npad, 32), jnp.float32)

  xg = _make_gather(epad, k, 16)(x16, src3)
  msg1 = _tc_msg(ea_pad, xg, nn1_w1, nn1_b1, nn1_w2, nn1_b2, nh=2, bt=2048)
  p1 = _make_scatter_add(epad, k, npad)(msg1, dst3, zacc)
  h1 = _tc_node(p1, x_pad, root1, bias1, root1, bias1,
                relu_out=True, bn=1024)

  h1g = _make_gather(epad, k, 32)(h1, src3)
  msg2 = _tc_msg(ea_pad, h1g, nn2_w1, nn2_b1, nn2_w2, nn2_b2, nh=32, bt=2048)
  p2 = _make_scatter_add(epad, k, npad)(msg2, dst3, zacc)
  out = _tc_node(p2, h1, root2, bias2, fc2_w, fc2_b,
                 relu_out=False, bn=1024)
  return out[:n]


# speedup vs baseline: 2.0604x; 2.0604x over previous
import functools

import jax
import jax.numpy as jnp
from jax import lax
from jax.experimental import pallas as pl
from jax.experimental.pallas import tpu as pltpu
from jax.experimental.pallas import tpu_sc as plsc

NC = 2
NS = 16
NW = NC * NS
C = 128


def _sc_mesh():
  return plsc.VectorSubcoreMesh(
      core_axis_name="c", subcore_axis_name="s", num_cores=NC, num_subcores=NS
  )


def _make_gather(epad, k, d, dtype=jnp.float32):
  @functools.partial(
      pl.kernel,
      out_type=jax.ShapeDtypeStruct((epad, d), dtype),
      mesh=_sc_mesh(),
      scratch_types=[
          pltpu.VMEM((C,), jnp.int32),
          pltpu.VMEM((C, d), dtype),
          pltpu.SemaphoreType.DMA,
      ],
      compiler_params=pltpu.CompilerParams(use_tc_tiling_on_sc=False),
  )
  def gather(table_hbm, idx_hbm, out_hbm, idx_v, rows_v, sem):
    cid = lax.axis_index("c")
    sid = lax.axis_index("s")
    w = cid * NS + sid

    def body(j, _):
      pltpu.sync_copy(idx_hbm.at[w, j], idx_v)
      pltpu.async_copy(table_hbm.at[idx_v], rows_v, sem).wait()
      base = (w * k + j) * C
      pltpu.sync_copy(rows_v, out_hbm.at[pl.ds(base, C), :])
      return 0

    lax.fori_loop(0, k, body, 0)

  return gather


def _make_scatter_add(epad, k, npad):
  rpt = npad // NS

  @functools.partial(
      pl.kernel,
      out_type=jax.ShapeDtypeStruct((NC, npad, 32), jnp.float32),
      mesh=_sc_mesh(),
      scratch_types=[
          pltpu.VMEM((C,), jnp.int32),
          pltpu.VMEM((C, 32), jnp.float32),
          pltpu.VMEM_SHARED((npad, 32), jnp.float32),
      ],
      compiler_params=pltpu.CompilerParams(use_tc_tiling_on_sc=False),
  )
  def scatter(msg_hbm, dst_hbm, zeros_hbm, out_hbm, idx_v, rows_v, acc):
    cid = lax.axis_index("c")
    sid = lax.axis_index("s")
    w = cid * NS + sid

    pltpu.sync_copy(zeros_hbm.at[pl.ds(sid * rpt, rpt), :],
                    acc.at[pl.ds(sid * rpt, rpt), :])
    plsc.subcore_barrier()

    def body(j, _):
      pltpu.sync_copy(dst_hbm.at[w, j], idx_v)
      base = (w * k + j) * C
      pltpu.sync_copy(msg_hbm.at[pl.ds(base, C), :], rows_v)
      pltpu.sync_copy(rows_v, acc.at[idx_v], add=True)
      return 0

    lax.fori_loop(0, k, body, 0)
    plsc.subcore_barrier()
    pltpu.sync_copy(acc.at[pl.ds(sid * rpt, rpt), :],
                    out_hbm.at[cid, pl.ds(sid * rpt, rpt), :])

  return scatter


def _msg_body(nh, ea_ref, hg_ref, w1_ref, b1_ref, w2m_ref, b2m_ref, k1_ref,
              k2_ref, out_ref):
  ea = ea_ref[...]
  t = jnp.maximum(
      jnp.dot(ea, w1_ref[...], preferred_element_type=jnp.float32)
      + b1_ref[...], 0.0)
  hg = hg_ref[...][:, :nh]
  u = (jnp.dot(t, k1_ref[...], preferred_element_type=jnp.float32)
       * jnp.dot(hg, k2_ref[...], preferred_element_type=jnp.float32))
  out_ref[...] = (
      jnp.dot(u, w2m_ref[...], preferred_element_type=jnp.float32)
      + jnp.dot(hg, b2m_ref[...], preferred_element_type=jnp.float32))


def _tc_msg(ea, hg, w1, b1, w2, b2, nh, bt):
  epad = ea.shape[0]
  grid = epad // bt
  w2m = w2.reshape(16, nh, 32).reshape(16 * nh, 32)
  b2m = b2.reshape(nh, 32)
  k1 = jnp.repeat(jnp.eye(16, dtype=jnp.float32), nh, axis=1)
  k2 = jnp.tile(jnp.eye(nh, dtype=jnp.float32), (1, 16))
  return pl.pallas_call(
      functools.partial(_msg_body, nh),
      grid=(grid,),
      in_specs=[
          pl.BlockSpec((bt, 2), lambda i: (i, 0)),
          pl.BlockSpec((bt, hg.shape[1]), lambda i: (i, 0)),
          pl.BlockSpec((2, 16), lambda i: (0, 0)),
          pl.BlockSpec((1, 16), lambda i: (0, 0)),
          pl.BlockSpec((16 * nh, 32), lambda i: (0, 0)),
          pl.BlockSpec((nh, 32), lambda i: (0, 0)),
          pl.BlockSpec((16, 16 * nh), lambda i: (0, 0)),
          pl.BlockSpec((nh, 16 * nh), lambda i: (0, 0)),
      ],
      out_specs=pl.BlockSpec((bt, 32), lambda i: (i, 0)),
      out_shape=jax.ShapeDtypeStruct((epad, 32), jnp.float32),
  )(ea, hg, w1, b1.reshape(1, -1), w2m, b2m, k1, k2)


def _node_body(relu_out, p_ref, h_ref, r_ref, b_ref, wo_ref, bo_ref, out_ref):
  agg = p_ref[0] + p_ref[1]
  h = jnp.maximum(
      agg + jnp.dot(h_ref[...], r_ref[...], preferred_element_type=jnp.float32)
      + b_ref[...], 0.0)
  if relu_out:
    out_ref[...] = h
  else:
    out_ref[...] = (
        jnp.dot(h, wo_ref[...], preferred_element_type=jnp.float32)
        + bo_ref[...])


def _tc_node(partials, h, root, bias, w_out, b_out, relu_out, bn):
  npad = h.shape[0]
  dh = h.shape[1]
  dout = 32 if relu_out else w_out.shape[1]
  return pl.pallas_call(
      functools.partial(_node_body, relu_out),
      grid=(npad // bn,),
      in_specs=[
          pl.BlockSpec((2, bn, 32), lambda i: (0, i, 0)),
          pl.BlockSpec((bn, dh), lambda i: (i, 0)),
          pl.BlockSpec((dh, 32), lambda i: (0, 0)),
          pl.BlockSpec((1, 32), lambda i: (0, 0)),
          pl.BlockSpec(w_out.shape, lambda i: (0, 0)),
          pl.BlockSpec((1, w_out.shape[1]), lambda i: (0, 0)),
      ],
      out_specs=pl.BlockSpec((bn, dout), lambda i: (i, 0)),
      out_shape=jax.ShapeDtypeStruct((npad, dout), jnp.float32),
  )(partials, h, root, bias.reshape(1, -1), w_out, b_out.reshape(1, -1))


def kernel(x, edge_index, edge_attr,
           nn1_w1, nn1_b1, nn1_w2, nn1_b2, root1, bias1,
           nn2_w1, nn2_b1, nn2_w2, nn2_b2, root2, bias2,
           fc2_w, fc2_b):
  n = x.shape[0]
  e = edge_attr.shape[0]
  k = -(-e // (NW * C))
  epad = NW * k * C
  npad = -(-(n + 1) // 1024) * 1024

  src = edge_index[0]
  dst = edge_index[1]
  src3 = jnp.zeros((epad,), jnp.int32).at[:e].set(src).reshape(NW, k, C)
  dst3 = jnp.full((epad,), n, jnp.int32).at[:e].set(dst).reshape(NW, k, C)
  ea_pad = jnp.zeros((epad, 2), jnp.float32).at[:e].set(edge_attr)
  x16 = jnp.zeros((n, 16), jnp.float32).at[:, :2].set(x)
  x_pad = jnp.zeros((npad, 2), jnp.float32).at[:n].set(x)
  zacc = jnp.zeros((npad, 32), jnp.float32)

  xg = _make_gather(epad, k, 16)(x16, src3)
  msg1 = _tc_msg(ea_pad, xg, nn1_w1, nn1_b1, nn1_w2, nn1_b2, nh=2, bt=2048)
  p1 = _make_scatter_add(epad, k, npad)(msg1, dst3, zacc)
  h1 = _tc_node(p1, x_pad, root1, bias1, root1, bias1,
                relu_out=True, bn=1024)

  h1g = _make_gather(epad, k, 32)(h1, src3)
  msg2 = _tc_msg(ea_pad, h1g, nn2_w1, nn2_b1, nn2_w2, nn2_b2, nh=32, bt=2048)
  p2 = _make_scatter_add(epad, k, npad)(msg2, dst3, zacc)
  out = _tc_node(p2, h1, root2, bias2, fc2_w, fc2_b,
                 relu_out=False, bn=1024)
  return out[:n]


# speedup vs baseline: 2.3836x; 1.1569x over previous
import functools

import jax
import jax.numpy as jnp
from jax import lax
from jax.experimental import pallas as pl
from jax.experimental.pallas import tpu as pltpu
from jax.experimental.pallas import tpu_sc as plsc

NC = 2
NS = 16
NW = NC * NS
C = 128


def _sc_mesh():
  return plsc.VectorSubcoreMesh(
      core_axis_name="c", subcore_axis_name="s", num_cores=NC, num_subcores=NS
  )


def _make_gather(epad, k, d, q=10):
  kq = k // q

  @functools.partial(
      pl.kernel,
      out_type=jax.ShapeDtypeStruct((epad, d), jnp.float32),
      mesh=_sc_mesh(),
      scratch_types=[
          pltpu.VMEM((k, C), jnp.int32),
          pltpu.VMEM((q * C, d), jnp.float32),
          pltpu.VMEM((q * C, d), jnp.float32),
          pltpu.SemaphoreType.DMA,
          pltpu.SemaphoreType.DMA,
          pltpu.SemaphoreType.DMA,
      ],
      compiler_params=pltpu.CompilerParams(use_tc_tiling_on_sc=False),
  )
  def gather(table_hbm, idx_hbm, out_hbm, idx_v, buf_a, buf_b, gsem, wsem_a,
             wsem_b):
    cid = lax.axis_index("c")
    sid = lax.axis_index("s")
    w = cid * NS + sid
    pltpu.sync_copy(idx_hbm.at[w], idx_v)
    bufs = (buf_a, buf_b)
    wsems = (wsem_a, wsem_b)
    for g in range(kq):
      buf = bufs[g % 2]
      wsem = wsems[g % 2]
      if g >= 2:
        pltpu.make_async_copy(buf, out_hbm.at[pl.ds(0, q * C), :], wsem).wait()

      def fire(j, _, g=g, buf=buf):
        pltpu.async_copy(table_hbm.at[idx_v.at[g * q + j]],
                         buf.at[pl.ds(j * C, C), :], gsem)
        return 0

      def drain(j, _, g=g, buf=buf):
        pltpu.make_async_copy(table_hbm.at[idx_v.at[g * q + j]],
                              buf.at[pl.ds(j * C, C), :], gsem).wait()
        return 0

      lax.fori_loop(0, q, fire, 0)
      lax.fori_loop(0, q, drain, 0)
      base = (w * k + g * q) * C
      pltpu.async_copy(buf, out_hbm.at[pl.ds(base, q * C), :], wsem)
    pltpu.make_async_copy(buf_a, out_hbm.at[pl.ds(0, q * C), :], wsem_a).wait()
    pltpu.make_async_copy(buf_b, out_hbm.at[pl.ds(0, q * C), :], wsem_b).wait()

  return gather


def _make_scatter_add(epad, k, npad, q=10):
  kq = k // q
  rpt = npad // NS

  @functools.partial(
      pl.kernel,
      out_type=jax.ShapeDtypeStruct((NC, npad, 32), jnp.float32),
      mesh=_sc_mesh(),
      scratch_types=[
          pltpu.VMEM((k, C), jnp.int32),
          pltpu.VMEM((q * C, 32), jnp.float32),
          pltpu.VMEM((q * C, 32), jnp.float32),
          pltpu.SemaphoreType.DMA,
          pltpu.SemaphoreType.DMA,
          pltpu.SemaphoreType.DMA,
          pltpu.VMEM_SHARED((npad, 32), jnp.float32),
      ],
      compiler_params=pltpu.CompilerParams(use_tc_tiling_on_sc=False),
  )
  def scatter(msg_hbm, dst_hbm, zeros_hbm, out_hbm, idx_v, buf_a, buf_b,
              lsem_a, lsem_b, ssem, acc):
    cid = lax.axis_index("c")
    sid = lax.axis_index("s")
    w = cid * NS + sid
    pltpu.sync_copy(dst_hbm.at[w], idx_v)
    pltpu.async_copy(msg_hbm.at[pl.ds(w * k * C, q * C), :], buf_a, lsem_a)
    pltpu.sync_copy(zeros_hbm.at[pl.ds(sid * rpt, rpt), :],
                    acc.at[pl.ds(sid * rpt, rpt), :])
    plsc.subcore_barrier()
    bufs = (buf_a, buf_b)
    lsems = (lsem_a, lsem_b)
    for g in range(kq):
      buf = bufs[g % 2]
      pltpu.make_async_copy(msg_hbm.at[pl.ds(0, q * C), :], buf,
                            lsems[g % 2]).wait()
      if g + 1 < kq:
        nbase = (w * k + (g + 1) * q) * C
        pltpu.async_copy(msg_hbm.at[pl.ds(nbase, q * C), :],
                         bufs[(g + 1) % 2], lsems[(g + 1) % 2])

      def fire(j, _, g=g, buf=buf):
        pltpu.async_copy(buf.at[pl.ds(j * C, C), :],
                         acc.at[idx_v.at[g * q + j]], ssem, add=True)
        return 0

      def drain(j, _, g=g, buf=buf):
        pltpu.make_async_copy(buf.at[pl.ds(j * C, C), :],
                              acc.at[idx_v.at[g * q + j]], ssem).wait()
        return 0

      lax.fori_loop(0, q, fire, 0)
      lax.fori_loop(0, q, drain, 0)
    plsc.subcore_barrier()
    pltpu.sync_copy(acc.at[pl.ds(sid * rpt, rpt), :],
                    out_hbm.at[cid, pl.ds(sid * rpt, rpt), :])

  return scatter


def _msg_body(nh, ea_ref, hg_ref, w1_ref, b1_ref, w2m_ref, b2m_ref, k1_ref,
              k2_ref, out_ref):
  ea = ea_ref[...]
  t = jnp.maximum(
      jnp.dot(ea, w1_ref[...], preferred_element_type=jnp.float32)
      + b1_ref[...], 0.0)
  hg = hg_ref[...][:, :nh]
  u = (jnp.dot(t, k1_ref[...], preferred_element_type=jnp.float32)
       * jnp.dot(hg, k2_ref[...], preferred_element_type=jnp.float32))
  out_ref[...] = (
      jnp.dot(u, w2m_ref[...], preferred_element_type=jnp.float32)
      + jnp.dot(hg, b2m_ref[...], preferred_element_type=jnp.float32))


def _tc_msg(ea, hg, w1, b1, w2, b2, nh, bt):
  epad = ea.shape[0]
  grid = epad // bt
  w2m = w2.reshape(16, nh, 32).reshape(16 * nh, 32)
  b2m = b2.reshape(nh, 32)
  k1 = jnp.repeat(jnp.eye(16, dtype=jnp.float32), nh, axis=1)
  k2 = jnp.tile(jnp.eye(nh, dtype=jnp.float32), (1, 16))
  return pl.pallas_call(
      functools.partial(_msg_body, nh),
      grid=(grid,),
      in_specs=[
          pl.BlockSpec((bt, 2), lambda i: (i, 0)),
          pl.BlockSpec((bt, hg.shape[1]), lambda i: (i, 0)),
          pl.BlockSpec((2, 16), lambda i: (0, 0)),
          pl.BlockSpec((1, 16), lambda i: (0, 0)),
          pl.BlockSpec((16 * nh, 32), lambda i: (0, 0)),
          pl.BlockSpec((nh, 32), lambda i: (0, 0)),
          pl.BlockSpec((16, 16 * nh), lambda i: (0, 0)),
          pl.BlockSpec((nh, 16 * nh), lambda i: (0, 0)),
      ],
      out_specs=pl.BlockSpec((bt, 32), lambda i: (i, 0)),
      out_shape=jax.ShapeDtypeStruct((epad, 32), jnp.float32),
  )(ea, hg, w1, b1.reshape(1, -1), w2m, b2m, k1, k2)


def _node_body(relu_out, p_ref, h_ref, r_ref, b_ref, wo_ref, bo_ref, out_ref):
  agg = p_ref[0] + p_ref[1]
  h = jnp.maximum(
      agg + jnp.dot(h_ref[...], r_ref[...], preferred_element_type=jnp.float32)
      + b_ref[...], 0.0)
  if relu_out:
    out_ref[...] = h
  else:
    out_ref[...] = (
        jnp.dot(h, wo_ref[...], preferred_element_type=jnp.float32)
        + bo_ref[...])


def _tc_node(partials, h, root, bias, w_out, b_out, relu_out, bn):
  npad = h.shape[0]
  dh = h.shape[1]
  dout = 32 if relu_out else w_out.shape[1]
  return pl.pallas_call(
      functools.partial(_node_body, relu_out),
      grid=(npad // bn,),
      in_specs=[
          pl.BlockSpec((2, bn, 32), lambda i: (0, i, 0)),
          pl.BlockSpec((bn, dh), lambda i: (i, 0)),
          pl.BlockSpec((dh, 32), lambda i: (0, 0)),
          pl.BlockSpec((1, 32), lambda i: (0, 0)),
          pl.BlockSpec(w_out.shape, lambda i: (0, 0)),
          pl.BlockSpec((1, w_out.shape[1]), lambda i: (0, 0)),
      ],
      out_specs=pl.BlockSpec((bn, dout), lambda i: (i, 0)),
      out_shape=jax.ShapeDtypeStruct((npad, dout), jnp.float32),
  )(partials, h, root, bias.reshape(1, -1), w_out, b_out.reshape(1, -1))


def kernel(x, edge_index, edge_attr,
           nn1_w1, nn1_b1, nn1_w2, nn1_b2, root1, bias1,
           nn2_w1, nn2_b1, nn2_w2, nn2_b2, root2, bias2,
           fc2_w, fc2_b):
  n = x.shape[0]
  e = edge_attr.shape[0]
  k = -(-e // (NW * C))
  epad = NW * k * C
  npad = -(-(n + 1) // 1024) * 1024

  src = edge_index[0]
  dst = edge_index[1]
  src3 = jnp.zeros((epad,), jnp.int32).at[:e].set(src).reshape(NW, k, C)
  dst3 = jnp.full((epad,), n, jnp.int32).at[:e].set(dst).reshape(NW, k, C)
  ea_pad = jnp.zeros((epad, 2), jnp.float32).at[:e].set(edge_attr)
  x16 = jnp.zeros((n, 16), jnp.float32).at[:, :2].set(x)
  x_pad = jnp.zeros((npad, 2), jnp.float32).at[:n].set(x)
  zacc = jnp.zeros((npad, 32), jnp.float32)

  xg = _make_gather(epad, k, 16)(x16, src3)
  msg1 = _tc_msg(ea_pad, xg, nn1_w1, nn1_b1, nn1_w2, nn1_b2, nh=2, bt=2048)
  p1 = _make_scatter_add(epad, k, npad)(msg1, dst3, zacc)
  h1 = _tc_node(p1, x_pad, root1, bias1, root1, bias1,
                relu_out=True, bn=1024)

  h1g = _make_gather(epad, k, 32)(h1, src3)
  msg2 = _tc_msg(ea_pad, h1g, nn2_w1, nn2_b1, nn2_w2, nn2_b2, nh=32, bt=2048)
  p2 = _make_scatter_add(epad, k, npad)(msg2, dst3, zacc)
  out = _tc_node(p2, h1, root2, bias2, fc2_w, fc2_b,
                 relu_out=False, bn=1024)
  return out[:n]
